# baseline (speedup 1.0000x reference)
import functools

import jax
import jax.numpy as jnp
from jax import lax
from jax.experimental import pallas as pl
from jax.experimental.pallas import tpu as pltpu
from jax.experimental.pallas import tpu_sc as plsc

_VOCAB = 100000
_D = 1024
_B = 4
_S = 4096
_NC = 2
_NS = 16
_NW = _NC * _NS
_ROWS = _B * _S
_PPW = _S // _NW
_C = 16
_NQ = _PPW // _C
_NITEM = _NQ * _B
_NBUF = 4

_mesh = plsc.VectorSubcoreMesh(core_axis_name="c", subcore_axis_name="s")


@functools.partial(
    pl.kernel,
    mesh=_mesh,
    out_type=jax.ShapeDtypeStruct((_ROWS, _D), jnp.float32),
    scratch_types=[
        pltpu.VMEM((_NQ, _B, _C), jnp.int32),
        [pltpu.VMEM((_C, _D), jnp.float32)] * _NBUF,
        [pltpu.VMEM((_C, _D), jnp.float32)] * 2,
        [pltpu.SemaphoreType.DMA] * _NBUF,
        [pltpu.SemaphoreType.DMA] * _NBUF,
        [pltpu.SemaphoreType.DMA] * 2,
    ],
)
def _embed(idx_hbm, tok_hbm, pe_hbm, out_hbm, idx_v, dbufs, pebufs,
           sin, sout, spe):
    wid = lax.axis_index("s") * _NC + lax.axis_index("c")
    pbase = wid * _PPW
    pltpu.sync_copy(idx_hbm.at[wid], idx_v)

    def start_g(q, b, k):
        pltpu.async_copy(tok_hbm.at[idx_v.at[q, b]], dbufs[k], sin[k])

    def wait_g(q, b, k):
        pltpu.make_async_copy(tok_hbm.at[idx_v.at[q, b]], dbufs[k],
                              sin[k]).wait()

    def start_pe(q, kp):
        pltpu.async_copy(pe_hbm.at[pl.ds(pbase + q * _C, _C)], pebufs[kp],
                         spe[kp])

    def wait_pe(q, kp):
        pltpu.make_async_copy(pe_hbm.at[pl.ds(pbase + q * _C, _C)],
                              pebufs[kp], spe[kp]).wait()

    def out_rows(q, b):
        return pl.ds(b * _S + pbase + q * _C, _C)

    def start_out(q, b, k):
        pltpu.async_copy(dbufs[k], out_hbm.at[out_rows(q, b)], sout[k])

    def wait_out(q, b, k):
        pltpu.make_async_copy(dbufs[k], out_hbm.at[out_rows(q, b)],
                              sout[k]).wait()

    def add_pe(k, kp):
        def half_row(t, carry):
            r = t // 2
            j0 = (t % 2) * (_D // 2)
            for j in range(_D // 32):
                plsc.addupdate(dbufs[k].at[r, pl.ds(j0 + j * 16, 16)],
                               pebufs[kp][r, pl.ds(j0 + j * 16, 16)])
            return carry
        lax.fori_loop(0, 2 * _C, half_row, 0)

    def body(q, b, kp, first_of_q, wait_o, start_next, start_next_pe):
        kn = (b + 2) % _NBUF
        if first_of_q:
            wait_pe(q, kp)
        wait_g(q, b, b)
        if wait_o:
            qo, bo = (q, b - 2) if b >= 2 else (q - 1, b + 2)
            wait_out(qo, bo, kn)
        if start_next:
            qn, bn = (q, b + 2) if b < 2 else (q + 1, b - 2)
            start_g(qn, bn, kn)
        if first_of_q and start_next_pe:
            start_pe(q + 1, (kp + 1) % 2)
        add_pe(b, kp)
        start_out(q, b, b)

    start_pe(0, 0)
    start_g(0, 0, 0)
    start_g(0, 1, 1)
    for q in range(2):
        for b in range(_B):
            body(q, b, q % 2, first_of_q=(b == 0),
                 wait_o=not (q == 0 and b < 2), start_next=True,
                 start_next_pe=True)

    def outer(q2, carry):
        for qoff in range(2):
            q = q2 * 2 + qoff
            for b in range(_B):
                body(q, b, qoff, first_of_q=(b == 0), wait_o=True,
                     start_next=True, start_next_pe=True)
        return carry

    lax.fori_loop(1, _NQ // 2 - 1, outer, 0)

    for q in range(_NQ - 2, _NQ):
        for b in range(_B):
            body(q, b, q % 2, first_of_q=(b == 0), wait_o=True,
                 start_next=(q + 1 < _NQ or b < 2),
                 start_next_pe=(q + 1 < _NQ))
    wait_out(_NQ - 1, _B - 2, _B - 2)
    wait_out(_NQ - 1, _B - 1, _B - 1)


def kernel(x, token_table, pe_table):
    idx = (x.reshape(_B, _NW, _NQ, _C).transpose(1, 2, 0, 3)
           .astype(jnp.int32))
    out = _embed(idx, token_table, pe_table)
    return out.reshape(_B, _S, _D)


# speedup vs baseline: 1.0051x; 1.0051x over previous
import functools

import jax
import jax.numpy as jnp
from jax import lax
from jax.experimental import pallas as pl
from jax.experimental.pallas import tpu as pltpu
from jax.experimental.pallas import tpu_sc as plsc

_VOCAB = 100000
_D = 1024
_B = 4
_S = 4096
_NC = 2
_NS = 16
_NW = _NC * _NS
_ROWS = _B * _S
_PPW = _S // _NW
_C = 16
_NQ = _PPW // _C
_NITEM = _NQ * _B
_NBUF = 4

_mesh = plsc.VectorSubcoreMesh(core_axis_name="c", subcore_axis_name="s")


@functools.partial(
    pl.kernel,
    mesh=_mesh,
    out_type=jax.ShapeDtypeStruct((_ROWS, _D), jnp.float32),
    scratch_types=[
        pltpu.VMEM((_NQ, _B, _C), jnp.int32),
        [pltpu.VMEM((_C, _D), jnp.float32)] * _NBUF,
        [pltpu.VMEM((_C, _D), jnp.float32)] * 2,
        [pltpu.SemaphoreType.DMA] * _NBUF,
        [pltpu.SemaphoreType.DMA] * _NBUF,
        [pltpu.SemaphoreType.DMA] * 2,
    ],
)
def _embed(idx_hbm, tok_hbm, pe_hbm, out_hbm, idx_v, dbufs, pebufs,
           sin, sout, spe):
    wid = lax.axis_index("s") * _NC + lax.axis_index("c")
    pbase = wid * _PPW
    pltpu.sync_copy(idx_hbm.at[wid], idx_v)

    def start_g(q, b, k):
        pltpu.async_copy(tok_hbm.at[idx_v.at[q, b]], dbufs[k], sin[k])

    def wait_g(q, b, k):
        pltpu.make_async_copy(tok_hbm.at[idx_v.at[q, b]], dbufs[k],
                              sin[k]).wait()

    def start_pe(q, kp):
        pltpu.async_copy(pe_hbm.at[pl.ds(pbase + q * _C, _C)], pebufs[kp],
                         spe[kp])

    def wait_pe(q, kp):
        pltpu.make_async_copy(pe_hbm.at[pl.ds(pbase + q * _C, _C)],
                              pebufs[kp], spe[kp]).wait()

    def out_rows(q, b):
        return pl.ds(b * _S + pbase + q * _C, _C)

    def start_out(q, b, k):
        pltpu.async_copy(dbufs[k], out_hbm.at[out_rows(q, b)], sout[k])

    def wait_out(q, b, k):
        pltpu.make_async_copy(dbufs[k], out_hbm.at[out_rows(q, b)],
                              sout[k]).wait()

    def add_pe(k, kp):
        @plsc.parallel_loop(0, 2 * _C)
        def _half_row(t):
            r = t // 2
            j0 = (t % 2) * (_D // 2)
            for j in range(_D // 32):
                plsc.addupdate(dbufs[k].at[r, pl.ds(j0 + j * 16, 16)],
                               pebufs[kp][r, pl.ds(j0 + j * 16, 16)])

    def body(q, b, kp, first_of_q, wait_o, start_next, start_next_pe):
        kn = (b + 2) % _NBUF
        if first_of_q:
            wait_pe(q, kp)
        wait_g(q, b, b)
        if wait_o:
            qo, bo = (q, b - 2) if b >= 2 else (q - 1, b + 2)
            wait_out(qo, bo, kn)
        if start_next:
            qn, bn = (q, b + 2) if b < 2 else (q + 1, b - 2)
            start_g(qn, bn, kn)
        if first_of_q and start_next_pe:
            start_pe(q + 1, (kp + 1) % 2)
        add_pe(b, kp)
        start_out(q, b, b)

    start_pe(0, 0)
    start_g(0, 0, 0)
    start_g(0, 1, 1)
    for q in range(2):
        for b in range(_B):
            body(q, b, q % 2, first_of_q=(b == 0),
                 wait_o=not (q == 0 and b < 2), start_next=True,
                 start_next_pe=True)

    def outer(q2, carry):
        for qoff in range(2):
            q = q2 * 2 + qoff
            for b in range(_B):
                body(q, b, qoff, first_of_q=(b == 0), wait_o=True,
                     start_next=True, start_next_pe=True)
        return carry

    lax.fori_loop(1, _NQ // 2 - 1, outer, 0)

    for q in range(_NQ - 2, _NQ):
        for b in range(_B):
            body(q, b, q % 2, first_of_q=(b == 0), wait_o=True,
                 start_next=(q + 1 < _NQ or b < 2),
                 start_next_pe=(q + 1 < _NQ))
    wait_out(_NQ - 1, _B - 2, _B - 2)
    wait_out(_NQ - 1, _B - 1, _B - 1)


def kernel(x, token_table, pe_table):
    idx = (x.reshape(_B, _NW, _NQ, _C).transpose(1, 2, 0, 3)
           .astype(jnp.int32))
    out = _embed(idx, token_table, pe_table)
    return out.reshape(_B, _S, _D)
